# Initial kernel scaffold; baseline (speedup 1.0000x reference)
import jax
import jax.numpy as jnp
from jax.experimental import pallas as pl


def kernel(inp_features, inp_positions, out_positions, voxel_size, kernel, bias):
    raise NotImplementedError("write your pallas kernel here")


# speedup vs baseline: 3.8057x; 3.8057x over previous
import functools

import jax
import jax.numpy as jnp
from jax import lax
from jax.experimental import pallas as pl
from jax.experimental.pallas import tpu as pltpu
from jax.experimental.pallas import tpu_sc as plsc

G = 48
S = G + 2
NKEY = S * S * S
NC, NS, L = 2, 16, 16
NW = NC * NS
TAB = 125184
SEG1 = TAB // NS
PW = 1664
NCH = PW // 128
NPAD = NW * PW
FP = 50176
D = 128
SENT = 50000
DUMMY_SLOT = NKEY
PAD_BASE = (S + 1) * S + 1

_OFFS = [dx * S * S + dy * S + dz
         for dx in (-1, 0, 1) for dy in (-1, 0, 1) for dz in (-1, 0, 1)]

_MESH = plsc.VectorSubcoreMesh(core_axis_name="c", subcore_axis_name="s",
                               num_cores=NC, num_subcores=NS)
_SC_PARAMS = pltpu.CompilerParams(use_tc_tiling_on_sc=False)


def _k1_body(keys_hbm, vals_hbm, table_hbm, kv, vv, fill, sem):
    cid = lax.axis_index("c")
    sid = lax.axis_index("s")

    @pl.when(cid == 0)
    def _():
        def fill_body(i, _):
            fill[pl.ds(i * L, L)] = jnp.full((L,), SENT, jnp.int32)
            return 0
        lax.fori_loop(0, SEG1 // L, fill_body, 0)
        pltpu.sync_copy(fill, table_hbm.at[pl.ds(sid * SEG1, SEG1)])
        plsc.subcore_barrier()

        per = NPAD // NS
        nch = per // 128
        pltpu.sync_copy(keys_hbm.at[pl.ds(sid * nch, nch), :], kv)
        pltpu.sync_copy(vals_hbm.at[pl.ds(sid * nch, nch), :], vv)

        def scat(ch, _):
            pltpu.async_copy(vv.at[ch], table_hbm.at[kv.at[ch]], sem).wait()
            return 0
        lax.fori_loop(0, nch, scat, 0)


def _k2_body(base_hbm, table_hbm, gidx_hbm, base_v, qv, tv, sem):
    cid = lax.axis_index("c")
    sid = lax.axis_index("s")
    w = sid * NC + cid
    pltpu.sync_copy(base_hbm.at[pl.ds(w * NCH, NCH), :], base_v)
    for j in range(27):
        off = _OFFS[j]

        def add_off(r, _):
            for l in range(D // L):
                sl = pl.ds(l * L, L)
                qv[r, sl] = base_v[r, sl] + off
            return 0
        lax.fori_loop(0, NCH, add_off, 0)

        def gath(ch, _):
            pltpu.async_copy(table_hbm.at[qv.at[ch]], tv.at[ch], sem).wait()
            return 0
        lax.fori_loop(0, NCH, gath, 0)

        jbase = j * FP

        def to_flat(r, _):
            for l in range(D // L):
                sl = pl.ds(l * L, L)
                tv[r, sl] = tv[r, sl] + jbase
            return 0
        lax.fori_loop(0, NCH, to_flat, 0)
        pltpu.sync_copy(tv, gidx_hbm.at[j, pl.ds(w * NCH, NCH), :])


def _k3_body(f_ref, w_ref, u_ref):
    u_ref[0] = jnp.dot(f_ref[...], w_ref[0],
                       preferred_element_type=jnp.float32)


def _k4_body(gidx_hbm, u2_hbm, bias_hbm, out_hbm,
             gidx_v, bias_v, acc, buf0, buf1, sem0, sem1):
    cid = lax.axis_index("c")
    sid = lax.axis_index("s")
    w = sid * NC + cid
    pltpu.sync_copy(bias_hbm, bias_v)
    for j in range(27):
        pltpu.sync_copy(gidx_hbm.at[j, pl.ds(w * NCH, NCH), :], gidx_v.at[j])

    bufs = (buf0, buf1)
    sems = (sem0, sem1)

    def chunk(c, _):
        def initr(r, _):
            for l in range(D // L):
                sl = pl.ds(l * L, L)
                acc[r, sl] = bias_v[sl]
            return 0
        lax.fori_loop(0, 128, initr, 0)

        pltpu.async_copy(u2_hbm.at[gidx_v.at[0, c]], bufs[0], sems[0])
        for j in range(27):
            buf = bufs[j % 2]
            pltpu.make_async_copy(u2_hbm.at[gidx_v.at[j, c]], buf,
                                  sems[j % 2]).wait()
            if j + 1 < 27:
                pltpu.async_copy(u2_hbm.at[gidx_v.at[j + 1, c]],
                                 bufs[(j + 1) % 2], sems[(j + 1) % 2])

            def addr(r, _):
                for l in range(D // L):
                    sl = pl.ds(l * L, L)
                    plsc.addupdate(acc.at[r, sl], buf[r, sl])
                return 0
            lax.fori_loop(0, 128, addr, 0)
        pltpu.sync_copy(acc, out_hbm.at[pl.ds(w * PW + c * 128, 128), :])
        return 0
    lax.fori_loop(0, NCH, chunk, 0)


_k1 = functools.partial(
    pl.kernel, _k1_body,
    out_type=jax.ShapeDtypeStruct((TAB,), jnp.int32),
    mesh=_MESH,
    compiler_params=_SC_PARAMS,
    scratch_types=[
        pltpu.VMEM((NPAD // NS // 128, 128), jnp.int32),
        pltpu.VMEM((NPAD // NS // 128, 128), jnp.int32),
        pltpu.VMEM((SEG1,), jnp.int32),
        pltpu.SemaphoreType.DMA,
    ],
)()

_k2 = functools.partial(
    pl.kernel, _k2_body,
    out_type=jax.ShapeDtypeStruct((27, NPAD // 128, 128), jnp.int32),
    mesh=_MESH,
    compiler_params=_SC_PARAMS,
    scratch_types=[
        pltpu.VMEM((NCH, 128), jnp.int32),
        pltpu.VMEM((NCH, 128), jnp.int32),
        pltpu.VMEM((NCH, 128), jnp.int32),
        pltpu.SemaphoreType.DMA,
    ],
)()

_k4 = functools.partial(
    pl.kernel, _k4_body,
    out_type=jax.ShapeDtypeStruct((NPAD, D), jnp.float32),
    mesh=_MESH,
    compiler_params=_SC_PARAMS,
    scratch_types=[
        pltpu.VMEM((27, NCH, 128), jnp.int32),
        pltpu.VMEM((D,), jnp.float32),
        pltpu.VMEM((128, D), jnp.float32),
        pltpu.VMEM((128, D), jnp.float32),
        pltpu.VMEM((128, D), jnp.float32),
        pltpu.SemaphoreType.DMA,
        pltpu.SemaphoreType.DMA,
    ],
)()

_k3 = pl.pallas_call(
    _k3_body,
    grid=(27, FP // 512),
    in_specs=[
        pl.BlockSpec((512, D), lambda j, i: (i, 0)),
        pl.BlockSpec((1, D, D), lambda j, i: (j, 0, 0)),
    ],
    out_specs=pl.BlockSpec((1, 512, D), lambda j, i: (j, i, 0)),
    out_shape=jax.ShapeDtypeStruct((27, FP, D), jnp.float32),
)


def kernel(inp_features, inp_positions, out_positions, voxel_size, kernel, bias):
    n_in = inp_features.shape[0]
    n_out = out_positions.shape[0]
    vs = jnp.asarray(voxel_size, inp_positions.dtype)
    ic = jnp.floor(inp_positions / vs).astype(jnp.int32)
    oc = jnp.floor(out_positions / vs).astype(jnp.int32)
    keys_in = ((ic[:, 0] + 1) * S + (ic[:, 1] + 1)) * S + (ic[:, 2] + 1)
    base_out = ((oc[:, 0] + 1) * S + (oc[:, 1] + 1)) * S + (oc[:, 2] + 1)

    keys_pad = jnp.full((NPAD,), DUMMY_SLOT, jnp.int32).at[:n_in].set(keys_in)
    vals = jnp.arange(NPAD, dtype=jnp.int32)
    base_pad = jnp.full((NPAD,), PAD_BASE, jnp.int32).at[:n_out].set(base_out)
    feat_pad = jnp.zeros((FP, D), jnp.float32).at[:n_in].set(inp_features)
    w27 = kernel.reshape(27, D, D).astype(jnp.float32)

    table = _k1(keys_pad.reshape(NPAD // 128, 128), vals.reshape(NPAD // 128, 128))
    gidx = _k2(base_pad.reshape(NPAD // 128, 128), table)
    u2 = _k3(feat_pad, w27)
    res = _k4(gidx, u2.reshape(27 * FP, D), bias.astype(jnp.float32))
    return res[:n_out]


# speedup vs baseline: 5.1293x; 1.3478x over previous
import functools

import numpy as np
import jax
import jax.numpy as jnp
from jax import lax
from jax.experimental import pallas as pl
from jax.experimental.pallas import tpu as pltpu
from jax.experimental.pallas import tpu_sc as plsc

G = 48
S = G + 2
NKEY = S * S * S
NC, NS, L = 2, 16, 16
NW = NC * NS
TAB = 125184
SEG1 = TAB // NS
PW = 1664
NCH = PW // 128
NPAD = NW * PW
FP = 50176
D = 128
SENT = 50000
DUMMY_SLOT = NKEY
PAD_BASE = (S + 1) * S + 1

_OFFS = [dx * S * S + dy * S + dz
         for dx in (-1, 0, 1) for dy in (-1, 0, 1) for dz in (-1, 0, 1)]

_BLK = np.empty((32,), np.int32)
_BLK[0::2] = np.arange(16)
_BLK[1::2] = 16 + np.arange(16)
_GPERM = np.concatenate([32 * l + _BLK for l in range(4)])

_MESH = plsc.VectorSubcoreMesh(core_axis_name="c", subcore_axis_name="s",
                               num_cores=NC, num_subcores=NS)
_SC_PARAMS = pltpu.CompilerParams(use_tc_tiling_on_sc=False,
                                  needs_layout_passes=False)


def _k1_body(keys_hbm, vals_hbm, table_hbm, kv, vv, fill, sem):
    cid = lax.axis_index("c")
    sid = lax.axis_index("s")

    @pl.when(cid == 0)
    def _():
        def fill_body(i, _):
            fill[pl.ds(i * L, L)] = jnp.full((L,), SENT, jnp.int32)
            return 0
        lax.fori_loop(0, SEG1 // L, fill_body, 0)
        pltpu.sync_copy(fill, table_hbm.at[pl.ds(sid * SEG1, SEG1)])
        plsc.subcore_barrier()

        per = NPAD // NS
        nch = per // 128
        pltpu.sync_copy(keys_hbm.at[pl.ds(sid * nch, nch), :], kv)
        pltpu.sync_copy(vals_hbm.at[pl.ds(sid * nch, nch), :], vv)

        def fire(ch, _):
            pltpu.async_copy(vv.at[ch], table_hbm.at[kv.at[ch]], sem)
            return 0
        lax.fori_loop(0, nch, fire, 0)

        def drain(ch, _):
            pltpu.make_async_copy(vv.at[ch], table_hbm.at[kv.at[ch]],
                                  sem).wait()
            return 0
        lax.fori_loop(0, nch, drain, 0)


def _k3_body(f_ref, w_ref, u_ref):
    u_ref[0] = jnp.dot(f_ref[...], w_ref[0],
                       preferred_element_type=jnp.float32).astype(jnp.bfloat16)


def _k4_body(base_hbm, table_hbm, u2_hbm, bias_hbm, out_hbm,
             base_v, qv, gidx_v, bias_v, acc, buf0, buf1,
             sem_t, sem0, sem1):
    cid = lax.axis_index("c")
    sid = lax.axis_index("s")
    w = sid * NC + cid
    pltpu.sync_copy(bias_hbm, bias_v)
    pltpu.sync_copy(base_hbm.at[pl.ds(w * NCH, NCH), :], base_v)

    def _fire_tab(i):
        def body(r, _):
            pltpu.async_copy(table_hbm.at[qv.at[i % 2, r]],
                             gidx_v.at[i, r], sem_t)
            return 0
        lax.fori_loop(0, NCH, body, 0)

    def _wait_tab(i):
        def body(r, _):
            pltpu.make_async_copy(table_hbm.at[qv.at[i % 2, r]],
                                  gidx_v.at[i, r], sem_t).wait()
            return 0
        lax.fori_loop(0, NCH, body, 0)

    def _to_flat(i):
        jbase = i * FP

        def body(r, _):
            for l in range(D // L):
                sl = pl.ds(l * L, L)
                gidx_v[i, r, sl] = gidx_v[i, r, sl] + jbase
            return 0
        lax.fori_loop(0, NCH, body, 0)

    for j in range(27):
        if j >= 2:
            _wait_tab(j - 2)
            _to_flat(j - 2)
        off = _OFFS[j]

        def qbody(r, _):
            for l in range(D // L):
                sl = pl.ds(l * L, L)
                qv[j % 2, r, sl] = base_v[r, sl] + off
            return 0
        lax.fori_loop(0, NCH, qbody, 0)
        _fire_tab(j)
    for j in (25, 26):
        _wait_tab(j)
        _to_flat(j)

    bufs = (buf0, buf1)
    sems = (sem0, sem1)

    def chunk(c, _):
        def initr(r, _):
            for l in range(D // L):
                sl = pl.ds(l * L, L)
                acc[r, sl] = bias_v[sl]
            return 0
        lax.fori_loop(0, 128, initr, 0)

        pltpu.async_copy(u2_hbm.at[gidx_v.at[0, c]], bufs[0], sems[0])
        for j in range(27):
            buf = bufs[j % 2]
            pltpu.make_async_copy(u2_hbm.at[gidx_v.at[j, c]], buf,
                                  sems[j % 2]).wait()
            if j + 1 < 27:
                pltpu.async_copy(u2_hbm.at[gidx_v.at[j + 1, c]],
                                 bufs[(j + 1) % 2], sems[(j + 1) % 2])

            def addr(r, _):
                for l in range(4):
                    v = buf[r, pl.ds(32 * l, 32)]
                    a, b = plsc.unpack(v, format=plsc.PackFormat.INTERLEAVED)
                    plsc.addupdate(acc.at[r, pl.ds(32 * l, L)], a)
                    plsc.addupdate(acc.at[r, pl.ds(32 * l + L, L)], b)
                return 0
            lax.fori_loop(0, 128, addr, 0)
        pltpu.sync_copy(acc, out_hbm.at[pl.ds(w * PW + c * 128, 128), :])
        return 0
    lax.fori_loop(0, NCH, chunk, 0)


_k1 = functools.partial(
    pl.kernel, _k1_body,
    out_type=jax.ShapeDtypeStruct((TAB,), jnp.int32),
    mesh=_MESH,
    compiler_params=_SC_PARAMS,
    scratch_types=[
        pltpu.VMEM((NPAD // NS // 128, 128), jnp.int32),
        pltpu.VMEM((NPAD // NS // 128, 128), jnp.int32),
        pltpu.VMEM((SEG1,), jnp.int32),
        pltpu.SemaphoreType.DMA,
    ],
)()

_k4 = functools.partial(
    pl.kernel, _k4_body,
    out_type=jax.ShapeDtypeStruct((NPAD, D), jnp.float32),
    mesh=_MESH,
    compiler_params=_SC_PARAMS,
    scratch_types=[
        pltpu.VMEM((NCH, 128), jnp.int32),
        pltpu.VMEM((2, NCH, 128), jnp.int32),
        pltpu.VMEM((27, NCH, 128), jnp.int32),
        pltpu.VMEM((D,), jnp.float32),
        pltpu.VMEM((128, D), jnp.float32),
        pltpu.VMEM((128, D), jnp.bfloat16),
        pltpu.VMEM((128, D), jnp.bfloat16),
        pltpu.SemaphoreType.DMA,
        pltpu.SemaphoreType.DMA,
        pltpu.SemaphoreType.DMA,
    ],
)()

_k3 = pl.pallas_call(
    _k3_body,
    grid=(FP // 1024, 27),
    in_specs=[
        pl.BlockSpec((1024, D), lambda i, j: (i, 0)),
        pl.BlockSpec((1, D, D), lambda i, j: (j, 0, 0)),
    ],
    out_specs=pl.BlockSpec((1, 1024, D), lambda i, j: (j, i, 0)),
    out_shape=jax.ShapeDtypeStruct((27, FP, D), jnp.bfloat16),
)


def kernel(inp_features, inp_positions, out_positions, voxel_size, kernel, bias):
    n_in = inp_features.shape[0]
    n_out = out_positions.shape[0]
    vs = jnp.asarray(voxel_size, inp_positions.dtype)
    ic = jnp.floor(inp_positions / vs).astype(jnp.int32)
    oc = jnp.floor(out_positions / vs).astype(jnp.int32)
    keys_in = ((ic[:, 0] + 1) * S + (ic[:, 1] + 1)) * S + (ic[:, 2] + 1)
    base_out = ((oc[:, 0] + 1) * S + (oc[:, 1] + 1)) * S + (oc[:, 2] + 1)

    keys_pad = jnp.full((NPAD,), DUMMY_SLOT, jnp.int32).at[:n_in].set(keys_in)
    vals = jnp.arange(NPAD, dtype=jnp.int32)
    base_pad = jnp.full((NPAD,), PAD_BASE, jnp.int32).at[:n_out].set(base_out)
    feat_pad = jnp.zeros((FP, D), jnp.float32).at[:n_in].set(inp_features)
    w27 = kernel.reshape(27, D, D).astype(jnp.float32)[:, :, _GPERM]

    table = _k1(keys_pad.reshape(NPAD // 128, 128), vals.reshape(NPAD // 128, 128))
    u2 = _k3(feat_pad, w27)
    res = _k4(base_pad.reshape(NPAD // 128, 128), table,
              u2.reshape(27 * FP, D), bias.astype(jnp.float32))
    return res[:n_out]


# speedup vs baseline: 5.9952x; 1.1688x over previous
import functools

import numpy as np
import jax
import jax.numpy as jnp
from jax import lax
from jax.experimental import pallas as pl
from jax.experimental.pallas import tpu as pltpu
from jax.experimental.pallas import tpu_sc as plsc

G = 48
S = G + 2
NKEY = S * S * S
NC, NS, L = 2, 16, 16
NW = NC * NS
TAB = 125184
SEG1 = TAB // NS
PW = 1664
NCH = PW // 128
NPAD = NW * PW
KPT = NPAD // NS
KCH = KPT // 128
N_IN = 50000
N_OUT = 50000
FP = 50176
D = 128
FBLK = 1024
NBLK = FP // FBLK
SENT = N_IN
DUMMY_SLOT = NKEY
PAD_BASE = (S + 1) * S + 1
BCH = N_OUT // 128
REM = N_OUT - BCH * 128

_OFFS = [dx * S * S + dy * S + dz
         for dx in (-1, 0, 1) for dy in (-1, 0, 1) for dz in (-1, 0, 1)]

_BLK = np.empty((32,), np.int32)
_BLK[0::2] = np.arange(16)
_BLK[1::2] = 16 + np.arange(16)
_GPERM = np.concatenate([32 * l + _BLK for l in range(4)])

_MESH = plsc.VectorSubcoreMesh(core_axis_name="c", subcore_axis_name="s",
                               num_cores=NC, num_subcores=NS)
_SC_PARAMS = pltpu.CompilerParams(use_tc_tiling_on_sc=False,
                                  needs_layout_passes=False)


def _k3_body(f_ref, w_ref, u_ref):
    i = pl.program_id(0)
    u = jnp.dot(f_ref[...], w_ref[0], preferred_element_type=jnp.float32)
    row = lax.broadcasted_iota(jnp.int32, (FBLK, 1), 0) + i * FBLK
    u = jnp.where(row < N_IN, u, 0.0)
    u_ref[...] = u.astype(jnp.bfloat16)


def _k4_body(keys_hbm, base_hbm, u2_hbm, bias_hbm, out_hbm,
             tab_sh, kv, vv, fill, base_v, qv, gidx_v, bias_v,
             acc, buf0, buf1, sem_t, sem0, sem1):
    cid = lax.axis_index("c")
    sid = lax.axis_index("s")
    w = sid * NC + cid

    def fill_body(i, _):
        fill[pl.ds(i * L, L)] = jnp.full((L,), SENT, jnp.int32)
        return 0
    lax.fori_loop(0, SEG1 // L, fill_body, 0)
    pltpu.sync_copy(fill, tab_sh.at[pl.ds(sid * SEG1, SEG1)])

    pltpu.sync_copy(keys_hbm.at[pl.ds(sid * KCH, KCH), :], kv)

    def vbody(ch, _):
        b = sid * KPT + ch * 128
        for m in range(8):
            vv[ch, pl.ds(m * L, L)] = lax.iota(jnp.int32, L) + (b + m * L)
        return 0
    lax.fori_loop(0, KCH, vbody, 0)
    plsc.subcore_barrier()

    def fire_sc(ch, _):
        pltpu.async_copy(vv.at[ch], tab_sh.at[kv.at[ch]], sem_t)
        return 0
    lax.fori_loop(0, KCH, fire_sc, 0)

    def drain_sc(ch, _):
        pltpu.make_async_copy(vv.at[ch], tab_sh.at[kv.at[ch]], sem_t).wait()
        return 0
    lax.fori_loop(0, KCH, drain_sc, 0)
    plsc.subcore_barrier()

    pltpu.sync_copy(bias_hbm, bias_v)
    pltpu.sync_copy(base_hbm.at[pl.ds(w * NCH, NCH), :], base_v)

    def _fire_tab(i):
        def body(r, _):
            pltpu.async_copy(tab_sh.at[qv.at[i % 2, r]],
                             gidx_v.at[i, r], sem_t)
            return 0
        lax.fori_loop(0, NCH, body, 0)

    def _wait_tab(i):
        def body(r, _):
            pltpu.make_async_copy(tab_sh.at[qv.at[i % 2, r]],
                                  gidx_v.at[i, r], sem_t).wait()
            return 0
        lax.fori_loop(0, NCH, body, 0)

    def _to_flat(i):
        jbase = i * FP

        def body(r, _):
            for l in range(D // L):
                sl = pl.ds(l * L, L)
                gidx_v[i, r, sl] = gidx_v[i, r, sl] + jbase
            return 0
        lax.fori_loop(0, NCH, body, 0)

    for j in range(27):
        if j >= 2:
            _wait_tab(j - 2)
            _to_flat(j - 2)
        off = _OFFS[j]

        def qbody(r, _):
            for l in range(D // L):
                sl = pl.ds(l * L, L)
                qv[j % 2, r, sl] = base_v[r, sl] + off
            return 0
        lax.fori_loop(0, NCH, qbody, 0)
        _fire_tab(j)
    for j in (25, 26):
        _wait_tab(j)
        _to_flat(j)

    bufs = (buf0, buf1)
    sems = (sem0, sem1)

    def chunk(c, _):
        s = w * NCH + c

        @pl.when(s <= BCH)
        def _():
            def initr(r, _):
                for l in range(D // L):
                    sl = pl.ds(l * L, L)
                    acc[r, sl] = bias_v[sl]
                return 0
            lax.fori_loop(0, 128, initr, 0)

            pltpu.async_copy(u2_hbm.at[gidx_v.at[0, c]], bufs[0], sems[0])
            for j in range(27):
                buf = bufs[j % 2]
                pltpu.make_async_copy(u2_hbm.at[gidx_v.at[j, c]], buf,
                                      sems[j % 2]).wait()
                if j + 1 < 27:
                    pltpu.async_copy(u2_hbm.at[gidx_v.at[j + 1, c]],
                                     bufs[(j + 1) % 2], sems[(j + 1) % 2])

                def addr(r, _):
                    for l in range(4):
                        v = buf[r, pl.ds(32 * l, 32)]
                        a, b = plsc.unpack(v,
                                           format=plsc.PackFormat.INTERLEAVED)
                        plsc.addupdate(acc.at[r, pl.ds(32 * l, L)], a)
                        plsc.addupdate(acc.at[r, pl.ds(32 * l + L, L)], b)
                    return 0
                lax.fori_loop(0, 128, addr, 0)

            @pl.when(s < BCH)
            def _():
                pltpu.sync_copy(acc,
                                out_hbm.at[pl.ds(w * PW + c * 128, 128), :])
            if REM:
                @pl.when(s == BCH)
                def _():
                    pltpu.sync_copy(acc.at[pl.ds(0, REM), :],
                                    out_hbm.at[pl.ds(BCH * 128, REM), :])
        return 0
    lax.fori_loop(0, NCH, chunk, 0)


_k4 = functools.partial(
    pl.kernel, _k4_body,
    out_type=jax.ShapeDtypeStruct((N_OUT, D), jnp.float32),
    mesh=_MESH,
    compiler_params=_SC_PARAMS,
    scratch_types=[
        pltpu.VMEM_SHARED((TAB,), jnp.int32),
        pltpu.VMEM((KCH, 128), jnp.int32),
        pltpu.VMEM((KCH, 128), jnp.int32),
        pltpu.VMEM((SEG1,), jnp.int32),
        pltpu.VMEM((NCH, 128), jnp.int32),
        pltpu.VMEM((2, NCH, 128), jnp.int32),
        pltpu.VMEM((27, NCH, 128), jnp.int32),
        pltpu.VMEM((D,), jnp.float32),
        pltpu.VMEM((128, D), jnp.float32),
        pltpu.VMEM((128, D), jnp.bfloat16),
        pltpu.VMEM((128, D), jnp.bfloat16),
        pltpu.SemaphoreType.DMA,
        pltpu.SemaphoreType.DMA,
        pltpu.SemaphoreType.DMA,
    ],
)()

_k3 = pl.pallas_call(
    _k3_body,
    grid=(NBLK, 27),
    in_specs=[
        pl.BlockSpec((FBLK, D), lambda i, j: (i, 0)),
        pl.BlockSpec((1, D, D), lambda i, j: (j, 0, 0)),
    ],
    out_specs=pl.BlockSpec((FBLK, D), lambda i, j: (j * NBLK + i, 0)),
    out_shape=jax.ShapeDtypeStruct((27 * FP, D), jnp.bfloat16),
)


def kernel(inp_features, inp_positions, out_positions, voxel_size, kernel, bias):
    n_in = inp_features.shape[0]
    n_out = out_positions.shape[0]
    vs = jnp.asarray(voxel_size, inp_positions.dtype)
    ic = jnp.floor(inp_positions / vs).astype(jnp.int32)
    oc = jnp.floor(out_positions / vs).astype(jnp.int32)
    keys_in = ((ic[:, 0] + 1) * S + (ic[:, 1] + 1)) * S + (ic[:, 2] + 1)
    base_out = ((oc[:, 0] + 1) * S + (oc[:, 1] + 1)) * S + (oc[:, 2] + 1)

    keys_pad = jnp.full((NPAD,), DUMMY_SLOT, jnp.int32).at[:n_in].set(keys_in)
    base_pad = jnp.full((NPAD,), PAD_BASE, jnp.int32).at[:n_out].set(base_out)
    w27 = kernel.reshape(27, D, D).astype(jnp.float32)[:, :, _GPERM]

    u2 = _k3(inp_features.astype(jnp.float32), w27)
    res = _k4(keys_pad.reshape(NPAD // 128, 128),
              base_pad.reshape(NPAD // 128, 128),
              u2, bias.astype(jnp.float32))
    return res


# speedup vs baseline: 6.0743x; 1.0132x over previous
import functools

import numpy as np
import jax
import jax.numpy as jnp
from jax import lax
from jax.experimental import pallas as pl
from jax.experimental.pallas import tpu as pltpu
from jax.experimental.pallas import tpu_sc as plsc

G = 48
S = G + 2
NKEY = S * S * S
NC, NS, L = 2, 16, 16
NW = NC * NS
TAB = 125184
SEG1 = TAB // NS
PW = 1664
NCH = PW // 128
NPAD = NW * PW
KPT = NPAD // NS
KCH = KPT // 128
N_IN = 50000
N_OUT = 50000
FP = 50176
D = 128
FBLK = 1024
NBLK = FP // FBLK
SENT = N_IN
DUMMY_SLOT = NKEY
PAD_BASE = (S + 1) * S + 1
BCH = N_OUT // 128
REM = N_OUT - BCH * 128

_OFFS = [dx * S * S + dy * S + dz
         for dx in (-1, 0, 1) for dy in (-1, 0, 1) for dz in (-1, 0, 1)]

_BLK = np.empty((32,), np.int32)
_BLK[0::2] = np.arange(16)
_BLK[1::2] = 16 + np.arange(16)
_GPERM = np.concatenate([32 * l + _BLK for l in range(4)])

_MESH = plsc.VectorSubcoreMesh(core_axis_name="c", subcore_axis_name="s",
                               num_cores=NC, num_subcores=NS)
_SC_PARAMS = pltpu.CompilerParams(use_tc_tiling_on_sc=False,
                                  needs_layout_passes=False)


def _k3_body(f_ref, w_ref, u_ref):
    i = pl.program_id(0)
    u = jnp.dot(f_ref[...], w_ref[0], preferred_element_type=jnp.float32)
    row = lax.broadcasted_iota(jnp.int32, (FBLK, 1), 0) + i * FBLK
    u = jnp.where(row < N_IN, u, 0.0)
    u_ref[...] = u.astype(jnp.bfloat16)


def _k4_body(keys_hbm, base_hbm, u2_hbm, bias_hbm, out_hbm,
             tab_sh, kv, vv, fill, base_v, qv, gidx_v, bias_v,
             acc, buf0, buf1, buf2, buf3, sem_t, sem0, sem1, sem2, sem3):
    cid = lax.axis_index("c")
    sid = lax.axis_index("s")
    w = sid * NC + cid

    def fill_body(i, _):
        fill[pl.ds(i * L, L)] = jnp.full((L,), SENT, jnp.int32)
        return 0
    lax.fori_loop(0, SEG1 // L, fill_body, 0)
    pltpu.sync_copy(fill, tab_sh.at[pl.ds(sid * SEG1, SEG1)])

    pltpu.sync_copy(keys_hbm.at[pl.ds(sid * KCH, KCH), :], kv)

    def vbody(ch, _):
        b = sid * KPT + ch * 128
        for m in range(8):
            vv[ch, pl.ds(m * L, L)] = lax.iota(jnp.int32, L) + (b + m * L)
        return 0
    lax.fori_loop(0, KCH, vbody, 0)
    plsc.subcore_barrier()

    def fire_sc(ch, _):
        pltpu.async_copy(vv.at[ch], tab_sh.at[kv.at[ch]], sem_t)
        return 0
    lax.fori_loop(0, KCH, fire_sc, 0)

    def drain_sc(ch, _):
        pltpu.make_async_copy(vv.at[ch], tab_sh.at[kv.at[ch]], sem_t).wait()
        return 0
    lax.fori_loop(0, KCH, drain_sc, 0)
    plsc.subcore_barrier()

    pltpu.sync_copy(bias_hbm, bias_v)
    pltpu.sync_copy(base_hbm.at[pl.ds(w * NCH, NCH), :], base_v)

    def _fire_tab(i):
        def body(r, _):
            pltpu.async_copy(tab_sh.at[qv.at[i % 2, r]],
                             gidx_v.at[i, r], sem_t)
            return 0
        lax.fori_loop(0, NCH, body, 0)

    def _wait_tab(i):
        def body(r, _):
            pltpu.make_async_copy(tab_sh.at[qv.at[i % 2, r]],
                                  gidx_v.at[i, r], sem_t).wait()
            return 0
        lax.fori_loop(0, NCH, body, 0)

    def _to_flat(i):
        jbase = i * FP

        def body(r, _):
            for l in range(D // L):
                sl = pl.ds(l * L, L)
                gidx_v[i, r, sl] = gidx_v[i, r, sl] + jbase
            return 0
        lax.fori_loop(0, NCH, body, 0)

    for j in range(27):
        if j >= 2:
            _wait_tab(j - 2)
            _to_flat(j - 2)
        off = _OFFS[j]

        def qbody(r, _):
            for l in range(D // L):
                sl = pl.ds(l * L, L)
                qv[j % 2, r, sl] = base_v[r, sl] + off
            return 0
        lax.fori_loop(0, NCH, qbody, 0)
        _fire_tab(j)
    for j in (25, 26):
        _wait_tab(j)
        _to_flat(j)

    bufs = (buf0, buf1, buf2, buf3)
    sems = (sem0, sem1, sem2, sem3)

    def chunk(c, _):
        s = w * NCH + c

        @pl.when(s <= BCH)
        def _():
            def initr(r, _):
                for l in range(D // L):
                    sl = pl.ds(l * L, L)
                    acc[r, sl] = bias_v[sl]
                return 0
            lax.fori_loop(0, 128, initr, 0)

            for p in range(3):
                pltpu.async_copy(u2_hbm.at[gidx_v.at[p, c]], bufs[p], sems[p])
            for j in range(27):
                buf = bufs[j % 4]
                pltpu.make_async_copy(u2_hbm.at[gidx_v.at[j, c]], buf,
                                      sems[j % 4]).wait()
                if j + 3 < 27:
                    pltpu.async_copy(u2_hbm.at[gidx_v.at[j + 3, c]],
                                     bufs[(j + 3) % 4], sems[(j + 3) % 4])

                def addr(r, _):
                    for l in range(4):
                        v = buf[r, pl.ds(32 * l, 32)]
                        a, b = plsc.unpack(v,
                                           format=plsc.PackFormat.INTERLEAVED)
                        plsc.addupdate(acc.at[r, pl.ds(32 * l, L)], a)
                        plsc.addupdate(acc.at[r, pl.ds(32 * l + L, L)], b)
                    return 0
                lax.fori_loop(0, 128, addr, 0)

            @pl.when(s < BCH)
            def _():
                pltpu.sync_copy(acc,
                                out_hbm.at[pl.ds(w * PW + c * 128, 128), :])
            if REM:
                @pl.when(s == BCH)
                def _():
                    pltpu.sync_copy(acc.at[pl.ds(0, REM), :],
                                    out_hbm.at[pl.ds(BCH * 128, REM), :])
        return 0
    lax.fori_loop(0, NCH, chunk, 0)


_k4 = functools.partial(
    pl.kernel, _k4_body,
    out_type=jax.ShapeDtypeStruct((N_OUT, D), jnp.float32),
    mesh=_MESH,
    compiler_params=_SC_PARAMS,
    scratch_types=[
        pltpu.VMEM_SHARED((TAB,), jnp.int32),
        pltpu.VMEM((KCH, 128), jnp.int32),
        pltpu.VMEM((KCH, 128), jnp.int32),
        pltpu.VMEM((SEG1,), jnp.int32),
        pltpu.VMEM((NCH, 128), jnp.int32),
        pltpu.VMEM((2, NCH, 128), jnp.int32),
        pltpu.VMEM((27, NCH, 128), jnp.int32),
        pltpu.VMEM((D,), jnp.float32),
        pltpu.VMEM((128, D), jnp.float32),
        pltpu.VMEM((128, D), jnp.bfloat16),
        pltpu.VMEM((128, D), jnp.bfloat16),
        pltpu.VMEM((128, D), jnp.bfloat16),
        pltpu.VMEM((128, D), jnp.bfloat16),
        pltpu.SemaphoreType.DMA,
        pltpu.SemaphoreType.DMA,
        pltpu.SemaphoreType.DMA,
        pltpu.SemaphoreType.DMA,
        pltpu.SemaphoreType.DMA,
    ],
)()

_k3 = pl.pallas_call(
    _k3_body,
    grid=(NBLK, 27),
    in_specs=[
        pl.BlockSpec((FBLK, D), lambda i, j: (i, 0)),
        pl.BlockSpec((1, D, D), lambda i, j: (j, 0, 0)),
    ],
    out_specs=pl.BlockSpec((FBLK, D), lambda i, j: (j * NBLK + i, 0)),
    out_shape=jax.ShapeDtypeStruct((27 * FP, D), jnp.bfloat16),
)


def kernel(inp_features, inp_positions, out_positions, voxel_size, kernel, bias):
    n_in = inp_features.shape[0]
    n_out = out_positions.shape[0]
    vs = jnp.asarray(voxel_size, inp_positions.dtype)
    ic = jnp.floor(inp_positions / vs).astype(jnp.int32)
    oc = jnp.floor(out_positions / vs).astype(jnp.int32)
    keys_in = ((ic[:, 0] + 1) * S + (ic[:, 1] + 1)) * S + (ic[:, 2] + 1)
    base_out = ((oc[:, 0] + 1) * S + (oc[:, 1] + 1)) * S + (oc[:, 2] + 1)

    keys_pad = jnp.full((NPAD,), DUMMY_SLOT, jnp.int32).at[:n_in].set(keys_in)
    base_pad = jnp.full((NPAD,), PAD_BASE, jnp.int32).at[:n_out].set(base_out)
    w27 = kernel.reshape(27, D, D).astype(jnp.float32)[:, :, _GPERM]

    u2 = _k3(inp_features.astype(jnp.float32), w27)
    res = _k4(keys_pad.reshape(NPAD // 128, 128),
              base_pad.reshape(NPAD // 128, 128),
              u2, bias.astype(jnp.float32))
    return res


# speedup vs baseline: 8.0085x; 1.3184x over previous
import functools

import numpy as np
import jax
import jax.numpy as jnp
from jax import lax
from jax.experimental import pallas as pl
from jax.experimental.pallas import tpu as pltpu
from jax.experimental.pallas import tpu_sc as plsc

G = 48
S = G + 2
NKEY = S * S * S
NC, NS, L = 2, 16, 16
NW = NC * NS
TAB = 125184
SEG1 = TAB // NS
PW = 1664
NCH = PW // 128
NPAD = NW * PW
KPT = NPAD // NS
KCH = KPT // 128
N_IN = 50000
N_OUT = 50000
FP = 50176
D = 128
FBLK = 3584
NBLK = FP // FBLK
SENT = N_IN
DUMMY_SLOT = NKEY
PAD_BASE = (S + 1) * S + 1
BCH = N_OUT // 128
REM = N_OUT - BCH * 128

_OFFS = [dx * S * S + dy * S + dz
         for dx in (-1, 0, 1) for dy in (-1, 0, 1) for dz in (-1, 0, 1)]

_BLK = np.empty((32,), np.int32)
_BLK[0::2] = np.arange(16)
_BLK[1::2] = 16 + np.arange(16)
_GPERM = np.concatenate([32 * l + _BLK for l in range(4)])

_MESH = plsc.VectorSubcoreMesh(core_axis_name="c", subcore_axis_name="s",
                               num_cores=NC, num_subcores=NS)
_SC_PARAMS = pltpu.CompilerParams(use_tc_tiling_on_sc=False,
                                  needs_layout_passes=False)


def _k3_body(f_ref, w_ref, u_ref):
    i = pl.program_id(0)
    u = jnp.dot(f_ref[...], w_ref[0], preferred_element_type=jnp.float32)
    row = lax.broadcasted_iota(jnp.int32, (FBLK, 1), 0) + i * FBLK
    u_ref[...] = jnp.where(row < N_IN, u, 0.0)


def _k4_body(keys_hbm, base_hbm, u2_hbm, bias_hbm, out_hbm,
             tab_sh, kv, vv, fill, base_v, qv, gidx_v, bias_v,
             acc, buf0, buf1, sem_t, sem0, sem1):
    cid = lax.axis_index("c")
    sid = lax.axis_index("s")
    w = sid * NC + cid

    def fill_body(i, _):
        fill[pl.ds(i * L, L)] = jnp.full((L,), SENT, jnp.int32)
        return 0
    lax.fori_loop(0, SEG1 // L, fill_body, 0)
    pltpu.sync_copy(fill, tab_sh.at[pl.ds(sid * SEG1, SEG1)])

    pltpu.sync_copy(keys_hbm.at[pl.ds(sid * KCH, KCH), :], kv)

    def vbody(ch, _):
        b = sid * KPT + ch * 128
        for m in range(8):
            vv[ch, pl.ds(m * L, L)] = lax.iota(jnp.int32, L) + (b + m * L)
        return 0
    lax.fori_loop(0, KCH, vbody, 0)
    plsc.subcore_barrier()

    def fire_sc(ch, _):
        pltpu.async_copy(vv.at[ch], tab_sh.at[kv.at[ch]], sem_t)
        return 0
    lax.fori_loop(0, KCH, fire_sc, 0)

    def drain_sc(ch, _):
        pltpu.make_async_copy(vv.at[ch], tab_sh.at[kv.at[ch]], sem_t).wait()
        return 0
    lax.fori_loop(0, KCH, drain_sc, 0)
    plsc.subcore_barrier()

    pltpu.sync_copy(bias_hbm, bias_v)
    pltpu.sync_copy(base_hbm.at[pl.ds(w * NCH, NCH), :], base_v)

    def _fire_tab(i):
        def body(r, _):
            pltpu.async_copy(tab_sh.at[qv.at[i % 2, r]],
                             gidx_v.at[i, r], sem_t)
            return 0
        lax.fori_loop(0, NCH, body, 0)

    def _wait_tab(i):
        def body(r, _):
            pltpu.make_async_copy(tab_sh.at[qv.at[i % 2, r]],
                                  gidx_v.at[i, r], sem_t).wait()
            return 0
        lax.fori_loop(0, NCH, body, 0)

    def _to_flat(i):
        jbase = i * FP

        def body(r, _):
            for l in range(D // L):
                sl = pl.ds(l * L, L)
                gidx_v[i, r, sl] = gidx_v[i, r, sl] + jbase
            return 0
        lax.fori_loop(0, NCH, body, 0)

    for j in range(27):
        if j >= 2:
            _wait_tab(j - 2)
            _to_flat(j - 2)
        off = _OFFS[j]

        def qbody(r, _):
            for l in range(D // L):
                sl = pl.ds(l * L, L)
                qv[j % 2, r, sl] = base_v[r, sl] + off
            return 0
        lax.fori_loop(0, NCH, qbody, 0)
        _fire_tab(j)
    for j in (25, 26):
        _wait_tab(j)
        _to_flat(j)

    bufs = (buf0, buf1)
    sems = (sem0, sem1)

    def chunk(c, _):
        s = w * NCH + c

        @pl.when(s <= BCH)
        def _():
            def initr(r, _):
                for l in range(D // L):
                    sl = pl.ds(l * L, L)
                    acc[r, sl] = bias_v[sl]
                return 0
            lax.fori_loop(0, 128, initr, 0)

            pltpu.async_copy(u2_hbm.at[gidx_v.at[0, c]], bufs[0], sems[0])
            for j in range(27):
                buf = bufs[j % 2]
                pltpu.make_async_copy(u2_hbm.at[gidx_v.at[j, c]], buf,
                                      sems[j % 2]).wait()
                if j + 1 < 27:
                    pltpu.async_copy(u2_hbm.at[gidx_v.at[j + 1, c]],
                                     bufs[(j + 1) % 2], sems[(j + 1) % 2])

                def addr(r, _):
                    for l in range(D // L):
                        sl = pl.ds(l * L, L)
                        plsc.addupdate(acc.at[r, sl], buf[r, sl])
                    return 0
                lax.fori_loop(0, 128, addr, 0)

            @pl.when(s < BCH)
            def _():
                pltpu.sync_copy(acc,
                                out_hbm.at[pl.ds(w * PW + c * 128, 128), :])
            if REM:
                @pl.when(s == BCH)
                def _():
                    pltpu.sync_copy(acc.at[pl.ds(0, REM), :],
                                    out_hbm.at[pl.ds(BCH * 128, REM), :])
        return 0
    lax.fori_loop(0, NCH, chunk, 0)


_k4 = functools.partial(
    pl.kernel, _k4_body,
    out_type=jax.ShapeDtypeStruct((N_OUT, D), jnp.float32),
    mesh=_MESH,
    compiler_params=_SC_PARAMS,
    scratch_types=[
        pltpu.VMEM_SHARED((TAB,), jnp.int32),
        pltpu.VMEM((KCH, 128), jnp.int32),
        pltpu.VMEM((KCH, 128), jnp.int32),
        pltpu.VMEM((SEG1,), jnp.int32),
        pltpu.VMEM((NCH, 128), jnp.int32),
        pltpu.VMEM((2, NCH, 128), jnp.int32),
        pltpu.VMEM((27, NCH, 128), jnp.int32),
        pltpu.VMEM((D,), jnp.float32),
        pltpu.VMEM((128, D), jnp.float32),
        pltpu.VMEM((128, D), jnp.float32),
        pltpu.VMEM((128, D), jnp.float32),
        pltpu.SemaphoreType.DMA,
        pltpu.SemaphoreType.DMA,
        pltpu.SemaphoreType.DMA,
    ],
)()

_k3 = pl.pallas_call(
    _k3_body,
    grid=(NBLK, 27),
    in_specs=[
        pl.BlockSpec((FBLK, D), lambda i, j: (i, 0)),
        pl.BlockSpec((1, D, D), lambda i, j: (j, 0, 0)),
    ],
    out_specs=pl.BlockSpec((FBLK, D), lambda i, j: (j * NBLK + i, 0)),
    out_shape=jax.ShapeDtypeStruct((27 * FP, D), jnp.float32),
)


def kernel(inp_features, inp_positions, out_positions, voxel_size, kernel, bias):
    n_in = inp_features.shape[0]
    n_out = out_positions.shape[0]
    vs = jnp.asarray(voxel_size, inp_positions.dtype)
    ic = jnp.floor(inp_positions / vs).astype(jnp.int32)
    oc = jnp.floor(out_positions / vs).astype(jnp.int32)
    keys_in = ((ic[:, 0] + 1) * S + (ic[:, 1] + 1)) * S + (ic[:, 2] + 1)
    base_out = ((oc[:, 0] + 1) * S + (oc[:, 1] + 1)) * S + (oc[:, 2] + 1)

    keys_pad = jnp.full((NPAD,), DUMMY_SLOT, jnp.int32).at[:n_in].set(keys_in)
    base_pad = jnp.full((NPAD,), PAD_BASE, jnp.int32).at[:n_out].set(base_out)
    w27 = kernel.reshape(27, D, D).astype(jnp.bfloat16)

    u2 = _k3(inp_features.astype(jnp.bfloat16), w27)
    res = _k4(keys_pad.reshape(NPAD // 128, 128),
              base_pad.reshape(NPAD // 128, 128),
              u2, bias.astype(jnp.float32))
    return res


# speedup vs baseline: 14.9569x; 1.8676x over previous
import functools

import numpy as np
import jax
import jax.numpy as jnp
from jax import lax
from jax.experimental import pallas as pl
from jax.experimental.pallas import tpu as pltpu
from jax.experimental.pallas import tpu_sc as plsc

G = 48
S = G + 2
NKEY = S * S * S
NC, NS, L = 2, 16, 16
NW = NC * NS
TAB = 125184
SEG1 = TAB // NS
PW = 1664
NCH = PW // 128
NPAD = NW * PW
KPT = NPAD // NS
KCH = KPT // 128
N_IN = 50000
N_OUT = 50000
FP = 50176
D = 128
FBLK = 3584
NBLK = FP // FBLK
SENT = N_IN
DUMMY_SLOT = NKEY
PAD_BASE = (S + 1) * S + 1
BCH = N_OUT // 128
REM = N_OUT - BCH * 128

_OFFS = [dx * S * S + dy * S + dz
         for dx in (-1, 0, 1) for dy in (-1, 0, 1) for dz in (-1, 0, 1)]

_BLK = np.empty((32,), np.int32)
_BLK[0::2] = np.arange(16)
_BLK[1::2] = 16 + np.arange(16)
_GPERM = np.concatenate([32 * l + _BLK for l in range(4)])

_MESH = plsc.VectorSubcoreMesh(core_axis_name="c", subcore_axis_name="s",
                               num_cores=NC, num_subcores=NS)
_SC_PARAMS = pltpu.CompilerParams(use_tc_tiling_on_sc=False,
                                  needs_layout_passes=False)


def _k3_body(f_ref, w_ref, u_ref):
    i = pl.program_id(0)
    u = jnp.dot(f_ref[...], w_ref[0], preferred_element_type=jnp.float32)
    row = lax.broadcasted_iota(jnp.int32, (FBLK, 1), 0) + i * FBLK
    u_ref[...] = jnp.where(row < N_IN, u, 0.0)


def _k4_body(keys_hbm, base_hbm, u2_hbm, bias_hbm, out_hbm,
             tab_sh, kv, vv, fill, base_v, qv, gidx_v, bias_v,
             acc, gbuf, cidx, slots, sem_t, sem_g):
    cid = lax.axis_index("c")
    sid = lax.axis_index("s")
    w = sid * NC + cid

    def fill_body(i, _):
        fill[pl.ds(i * L, L)] = jnp.full((L,), SENT, jnp.int32)
        return 0
    lax.fori_loop(0, SEG1 // L, fill_body, 0)
    pltpu.sync_copy(fill, tab_sh.at[pl.ds(sid * SEG1, SEG1)])

    pltpu.sync_copy(keys_hbm.at[pl.ds(sid * KCH, KCH), :], kv)

    def vbody(ch, _):
        b = sid * KPT + ch * 128
        for m in range(8):
            vv[ch, pl.ds(m * L, L)] = lax.iota(jnp.int32, L) + (b + m * L)
        return 0
    lax.fori_loop(0, KCH, vbody, 0)
    plsc.subcore_barrier()

    def fire_sc(ch, _):
        pltpu.async_copy(vv.at[ch], tab_sh.at[kv.at[ch]], sem_t)
        return 0
    lax.fori_loop(0, KCH, fire_sc, 0)

    def drain_sc(ch, _):
        pltpu.make_async_copy(vv.at[ch], tab_sh.at[kv.at[ch]], sem_t).wait()
        return 0
    lax.fori_loop(0, KCH, drain_sc, 0)
    plsc.subcore_barrier()

    pltpu.sync_copy(bias_hbm, bias_v)
    pltpu.sync_copy(base_hbm.at[pl.ds(w * NCH, NCH), :], base_v)

    def _fire_tab(i):
        def body(r, _):
            pltpu.async_copy(tab_sh.at[qv.at[i % 2, r]],
                             gidx_v.at[i, r], sem_t)
            return 0
        lax.fori_loop(0, NCH, body, 0)

    def _wait_tab(i):
        def body(r, _):
            pltpu.make_async_copy(tab_sh.at[qv.at[i % 2, r]],
                                  gidx_v.at[i, r], sem_t).wait()
            return 0
        lax.fori_loop(0, NCH, body, 0)

    def _to_flat(i):
        jbase = i * FP

        def body(r, _):
            for l in range(D // L):
                sl = pl.ds(l * L, L)
                gidx_v[i, r, sl] = gidx_v[i, r, sl] + jbase
            return 0
        lax.fori_loop(0, NCH, body, 0)

    for j in range(27):
        if j >= 2:
            _wait_tab(j - 2)
            _to_flat(j - 2)
        off = _OFFS[j]

        def qbody(r, _):
            for l in range(D // L):
                sl = pl.ds(l * L, L)
                qv[j % 2, r, sl] = base_v[r, sl] + off
            return 0
        lax.fori_loop(0, NCH, qbody, 0)
        _fire_tab(j)
    for j in (25, 26):
        _wait_tab(j)
        _to_flat(j)

    def chunk(c, _):
        s = w * NCH + c

        @pl.when(s <= BCH)
        def _():
            def initr(r, _):
                for l in range(D // L):
                    sl = pl.ds(l * L, L)
                    acc[r, sl] = bias_v[sl]
                return 0
            lax.fori_loop(0, 128, initr, 0)

            def cjb(j, ptr):
                jinv = j * FP + SENT
                for g in range(D // L):
                    sl = pl.ds(g * L, L)
                    tv = gidx_v[j, c, sl]
                    m = tv != jinv
                    plsc.store_compressed(cidx.at[pl.ds(ptr, L)], tv, mask=m)
                    slv = lax.iota(jnp.int32, L) + (g * L)
                    plsc.store_compressed(slots.at[pl.ds(ptr, L)], slv, mask=m)
                    ptr = ptr + jnp.sum(m.astype(jnp.int32))
                return ptr
            nv = lax.fori_loop(0, 27, cjb, 0)

            for k in range(8):
                cidx[pl.ds(nv + k * L, L)] = jnp.full((L,), SENT, jnp.int32)
                slots[pl.ds(nv + k * L, L)] = jnp.zeros((L,), jnp.int32)
            nb = (nv + 127) // 128

            @pl.when(nb > 0)
            def _():
                pltpu.async_copy(u2_hbm.at[cidx.at[pl.ds(0, 128)]],
                                 gbuf.at[0], sem_g)

                def batch(b, _):
                    bb = b % 2
                    pltpu.make_async_copy(
                        u2_hbm.at[cidx.at[pl.ds(b * 128, 128)]],
                        gbuf.at[bb], sem_g).wait()

                    @pl.when(b + 1 < nb)
                    def _():
                        pltpu.async_copy(
                            u2_hbm.at[cidx.at[pl.ds((b + 1) * 128, 128)]],
                            gbuf.at[(b + 1) % 2], sem_g)
                    def arow(p, _):
                        sv = slots[pl.ds(b * 128 + (p // L) * L, L)]
                        kvec = jnp.zeros((L,), jnp.int32) + (p % L)
                        sp = sv[kvec]
                        for l in range(D // L):
                            sl2 = pl.ds(l * L, L)
                            colv = lax.iota(jnp.int32, L) + l * L
                            plsc.addupdate_scatter(acc, [sp, colv],
                                                   gbuf[bb, p, sl2])
                        return 0
                    lax.fori_loop(0, 128, arow, 0)
                    return 0
                lax.fori_loop(0, nb, batch, 0)

            @pl.when(s < BCH)
            def _():
                pltpu.sync_copy(acc,
                                out_hbm.at[pl.ds(w * PW + c * 128, 128), :])
            if REM:
                @pl.when(s == BCH)
                def _():
                    pltpu.sync_copy(acc.at[pl.ds(0, REM), :],
                                    out_hbm.at[pl.ds(BCH * 128, REM), :])
        return 0
    lax.fori_loop(0, NCH, chunk, 0)


_k4 = functools.partial(
    pl.kernel, _k4_body,
    out_type=jax.ShapeDtypeStruct((N_OUT, D), jnp.float32),
    mesh=_MESH,
    compiler_params=_SC_PARAMS,
    scratch_types=[
        pltpu.VMEM_SHARED((TAB,), jnp.int32),
        pltpu.VMEM((KCH, 128), jnp.int32),
        pltpu.VMEM((KCH, 128), jnp.int32),
        pltpu.VMEM((SEG1,), jnp.int32),
        pltpu.VMEM((NCH, 128), jnp.int32),
        pltpu.VMEM((2, NCH, 128), jnp.int32),
        pltpu.VMEM((27, NCH, 128), jnp.int32),
        pltpu.VMEM((D,), jnp.float32),
        pltpu.VMEM((128, D), jnp.float32),
        pltpu.VMEM((2, 128, D), jnp.float32),
        pltpu.VMEM((28 * 128,), jnp.int32),
        pltpu.VMEM((28 * 128,), jnp.int32),
        pltpu.SemaphoreType.DMA,
        pltpu.SemaphoreType.DMA,
    ],
)()

_k3 = pl.pallas_call(
    _k3_body,
    grid=(NBLK, 27),
    in_specs=[
        pl.BlockSpec((FBLK, D), lambda i, j: (i, 0)),
        pl.BlockSpec((1, D, D), lambda i, j: (j, 0, 0)),
    ],
    out_specs=pl.BlockSpec((FBLK, D), lambda i, j: (j * NBLK + i, 0)),
    out_shape=jax.ShapeDtypeStruct((27 * FP, D), jnp.float32),
)


def kernel(inp_features, inp_positions, out_positions, voxel_size, kernel, bias):
    n_in = inp_features.shape[0]
    n_out = out_positions.shape[0]
    vs = jnp.asarray(voxel_size, inp_positions.dtype)
    ic = jnp.floor(inp_positions / vs).astype(jnp.int32)
    oc = jnp.floor(out_positions / vs).astype(jnp.int32)
    keys_in = ((ic[:, 0] + 1) * S + (ic[:, 1] + 1)) * S + (ic[:, 2] + 1)
    base_out = ((oc[:, 0] + 1) * S + (oc[:, 1] + 1)) * S + (oc[:, 2] + 1)

    keys_pad = jnp.full((NPAD,), DUMMY_SLOT, jnp.int32).at[:n_in].set(keys_in)
    base_pad = jnp.full((NPAD,), PAD_BASE, jnp.int32).at[:n_out].set(base_out)
    w27 = kernel.reshape(27, D, D).astype(jnp.bfloat16)

    u2 = _k3(inp_features.astype(jnp.bfloat16), w27)
    res = _k4(keys_pad.reshape(NPAD // 128, 128),
              base_pad.reshape(NPAD // 128, 128),
              u2, bias.astype(jnp.float32))
    return res


# speedup vs baseline: 16.3310x; 1.0919x over previous
import functools

import numpy as np
import jax
import jax.numpy as jnp
from jax import lax
from jax.experimental import pallas as pl
from jax.experimental.pallas import tpu as pltpu
from jax.experimental.pallas import tpu_sc as plsc

G = 48
S = G + 2
NKEY = S * S * S
NC, NS, L = 2, 16, 16
NW = NC * NS
TAB = 125184
SEG1 = TAB // NS
PW = 1664
NCH = PW // 128
NPAD = NW * PW
KPT = NPAD // NS
KCH = KPT // 128
N_IN = 50000
N_OUT = 50000
FP = 50176
D = 128
FBLK = 7168
NBLK = FP // FBLK
SENT = N_IN
DUMMY_SLOT = NKEY
PAD_BASE = (S + 1) * S + 1
BCH = N_OUT // 128
REM = N_OUT - BCH * 128

_OFFS = [dx * S * S + dy * S + dz
         for dx in (-1, 0, 1) for dy in (-1, 0, 1) for dz in (-1, 0, 1)]

_BLK = np.empty((32,), np.int32)
_BLK[0::2] = np.arange(16)
_BLK[1::2] = 16 + np.arange(16)
_GPERM = np.concatenate([32 * l + _BLK for l in range(4)])

_MESH = plsc.VectorSubcoreMesh(core_axis_name="c", subcore_axis_name="s",
                               num_cores=NC, num_subcores=NS)
_SC_PARAMS = pltpu.CompilerParams(use_tc_tiling_on_sc=False,
                                  needs_layout_passes=False)


def _k3_body(f_ref, w_ref, u_ref):
    i = pl.program_id(0)
    u = jnp.dot(f_ref[...], w_ref[0], preferred_element_type=jnp.float32)
    row = lax.broadcasted_iota(jnp.int32, (FBLK, 1), 0) + i * FBLK
    u_ref[...] = jnp.where(row < N_IN, u, 0.0)


def _k4_body(keys_hbm, base_hbm, u2_hbm, bias_hbm, out_hbm,
             tab_sh, kv, vv, fill, base_v, qv, gidx_v, bias_v,
             acc, gbuf, cidx, slots, sem_t, sem_g):
    cid = lax.axis_index("c")
    sid = lax.axis_index("s")
    w = sid * NC + cid

    def fill_body(i, _):
        fill[pl.ds(i * L, L)] = jnp.full((L,), SENT, jnp.int32)
        return 0
    lax.fori_loop(0, SEG1 // L, fill_body, 0)
    pltpu.sync_copy(fill, tab_sh.at[pl.ds(sid * SEG1, SEG1)])

    pltpu.sync_copy(keys_hbm.at[pl.ds(sid * KCH, KCH), :], kv)

    def vbody(ch, _):
        b = sid * KPT + ch * 128
        for m in range(8):
            vv[ch, pl.ds(m * L, L)] = lax.iota(jnp.int32, L) + (b + m * L)
        return 0
    lax.fori_loop(0, KCH, vbody, 0)
    plsc.subcore_barrier()

    def fire_sc(ch, _):
        pltpu.async_copy(vv.at[ch], tab_sh.at[kv.at[ch]], sem_t)
        return 0
    lax.fori_loop(0, KCH, fire_sc, 0)

    def drain_sc(ch, _):
        pltpu.make_async_copy(vv.at[ch], tab_sh.at[kv.at[ch]], sem_t).wait()
        return 0
    lax.fori_loop(0, KCH, drain_sc, 0)
    plsc.subcore_barrier()

    pltpu.sync_copy(bias_hbm, bias_v)

    def bload(c2, _):
        pltpu.sync_copy(base_hbm.at[pl.ds(w + NW * c2, 1), :],
                        base_v.at[pl.ds(c2, 1), :])
        return 0
    lax.fori_loop(0, NCH, bload, 0)

    def _fire_tab(i):
        def body(r, _):
            pltpu.async_copy(tab_sh.at[qv.at[i % 2, r]],
                             gidx_v.at[i, r], sem_t)
            return 0
        lax.fori_loop(0, NCH, body, 0)

    def _wait_tab(i):
        def body(r, _):
            pltpu.make_async_copy(tab_sh.at[qv.at[i % 2, r]],
                                  gidx_v.at[i, r], sem_t).wait()
            return 0
        lax.fori_loop(0, NCH, body, 0)

    def _to_flat(i):
        jbase = i * FP

        def body(r, _):
            for l in range(D // L):
                sl = pl.ds(l * L, L)
                gidx_v[i, r, sl] = gidx_v[i, r, sl] + jbase
            return 0
        lax.fori_loop(0, NCH, body, 0)

    for j in range(27):
        if j >= 2:
            _wait_tab(j - 2)
            _to_flat(j - 2)
        off = _OFFS[j]

        def qbody(r, _):
            for l in range(D // L):
                sl = pl.ds(l * L, L)
                qv[j % 2, r, sl] = base_v[r, sl] + off
            return 0
        lax.fori_loop(0, NCH, qbody, 0)
        _fire_tab(j)
    for j in (25, 26):
        _wait_tab(j)
        _to_flat(j)

    def chunk(c, _):
        s = w + NW * c

        @pl.when(s <= BCH)
        def _():
            def initr(r, _):
                for l in range(D // L):
                    sl = pl.ds(l * L, L)
                    acc[r, sl] = bias_v[sl]
                return 0
            lax.fori_loop(0, 128, initr, 0)

            def cjb(j, ptr):
                jinv = j * FP + SENT
                for g in range(D // L):
                    sl = pl.ds(g * L, L)
                    tv = gidx_v[j, c, sl]
                    m = tv != jinv
                    plsc.store_compressed(cidx.at[pl.ds(ptr, L)], tv, mask=m)
                    slv = lax.iota(jnp.int32, L) + (g * L)
                    plsc.store_compressed(slots.at[pl.ds(ptr, L)], slv, mask=m)
                    ptr = ptr + jnp.sum(m.astype(jnp.int32))
                return ptr
            nv = lax.fori_loop(0, 27, cjb, 0)

            for k in range(8):
                cidx[pl.ds(nv + k * L, L)] = jnp.full((L,), SENT, jnp.int32)
                slots[pl.ds(nv + k * L, L)] = jnp.zeros((L,), jnp.int32)
            nb = (nv + 127) // 128

            @pl.when(nb > 0)
            def _():
                pltpu.async_copy(u2_hbm.at[cidx.at[pl.ds(0, 128)]],
                                 gbuf.at[0], sem_g)

                def batch(b, _):
                    bb = b % 2
                    pltpu.make_async_copy(
                        u2_hbm.at[cidx.at[pl.ds(b * 128, 128)]],
                        gbuf.at[bb], sem_g).wait()

                    @pl.when(b + 1 < nb)
                    def _():
                        pltpu.async_copy(
                            u2_hbm.at[cidx.at[pl.ds((b + 1) * 128, 128)]],
                            gbuf.at[(b + 1) % 2], sem_g)
                    def arow(p, _):
                        sv = slots[pl.ds(b * 128 + (p // L) * L, L)]
                        kvec = jnp.zeros((L,), jnp.int32) + (p % L)
                        sp = sv[kvec]
                        for l in range(D // L):
                            sl2 = pl.ds(l * L, L)
                            colv = lax.iota(jnp.int32, L) + l * L
                            plsc.addupdate_scatter(acc, [sp, colv],
                                                   gbuf[bb, p, sl2])
                        return 0
                    lax.fori_loop(0, 128, arow, 0)
                    return 0
                lax.fori_loop(0, nb, batch, 0)

            @pl.when(s < BCH)
            def _():
                pltpu.sync_copy(acc, out_hbm.at[pl.ds(s * 128, 128), :])
            if REM:
                @pl.when(s == BCH)
                def _():
                    pltpu.sync_copy(acc.at[pl.ds(0, REM), :],
                                    out_hbm.at[pl.ds(BCH * 128, REM), :])
        return 0
    lax.fori_loop(0, NCH, chunk, 0)


_k4 = functools.partial(
    pl.kernel, _k4_body,
    out_type=jax.ShapeDtypeStruct((N_OUT, D), jnp.float32),
    mesh=_MESH,
    compiler_params=_SC_PARAMS,
    scratch_types=[
        pltpu.VMEM_SHARED((TAB,), jnp.int32),
        pltpu.VMEM((KCH, 128), jnp.int32),
        pltpu.VMEM((KCH, 128), jnp.int32),
        pltpu.VMEM((SEG1,), jnp.int32),
        pltpu.VMEM((NCH, 128), jnp.int32),
        pltpu.VMEM((2, NCH, 128), jnp.int32),
        pltpu.VMEM((27, NCH, 128), jnp.int32),
        pltpu.VMEM((D,), jnp.float32),
        pltpu.VMEM((128, D), jnp.float32),
        pltpu.VMEM((2, 128, D), jnp.float32),
        pltpu.VMEM((28 * 128,), jnp.int32),
        pltpu.VMEM((28 * 128,), jnp.int32),
        pltpu.SemaphoreType.DMA,
        pltpu.SemaphoreType.DMA,
    ],
)()

_k3 = pl.pallas_call(
    _k3_body,
    grid=(NBLK, 27),
    in_specs=[
        pl.BlockSpec((FBLK, D), lambda i, j: (i, 0)),
        pl.BlockSpec((1, D, D), lambda i, j: (j, 0, 0)),
    ],
    out_specs=pl.BlockSpec((FBLK, D), lambda i, j: (j * NBLK + i, 0)),
    out_shape=jax.ShapeDtypeStruct((27 * FP, D), jnp.float32),
)


def kernel(inp_features, inp_positions, out_positions, voxel_size, kernel, bias):
    n_in = inp_features.shape[0]
    n_out = out_positions.shape[0]
    vs = jnp.asarray(voxel_size, inp_positions.dtype)
    ic = jnp.floor(inp_positions / vs).astype(jnp.int32)
    oc = jnp.floor(out_positions / vs).astype(jnp.int32)
    keys_in = ((ic[:, 0] + 1) * S + (ic[:, 1] + 1)) * S + (ic[:, 2] + 1)
    base_out = ((oc[:, 0] + 1) * S + (oc[:, 1] + 1)) * S + (oc[:, 2] + 1)

    keys_pad = jnp.full((NPAD,), DUMMY_SLOT, jnp.int32).at[:n_in].set(keys_in)
    base_pad = jnp.full((NPAD,), PAD_BASE, jnp.int32).at[:n_out].set(base_out)
    w27 = kernel.reshape(27, D, D).astype(jnp.bfloat16)

    u2 = _k3(inp_features.astype(jnp.bfloat16), w27)
    res = _k4(keys_pad.reshape(NPAD // 128, 128),
              base_pad.reshape(NPAD // 128, 128),
              u2, bias.astype(jnp.float32))
    return res
